# speedup vs baseline: 1.4392x; 1.4392x over previous
import functools

import jax
import jax.numpy as jnp
from jax import lax
from jax.experimental import pallas as pl
from jax.experimental.pallas import tpu as pltpu
from jax.experimental.pallas import tpu_sc as plsc

D = 128
H = 64
CHUNK = 128
NW = 32


def _absdiff_sc(x, src, dst, pad_e):
    chunks_per_worker = pad_e // (NW * CHUNK)
    mesh = plsc.VectorSubcoreMesh(core_axis_name="c", subcore_axis_name="s")

    @functools.partial(
        pl.kernel,
        mesh=mesh,
        out_type=jax.ShapeDtypeStruct((pad_e, D), jnp.float32),
        scratch_types=[
            pltpu.VMEM((CHUNK,), jnp.int32),
            pltpu.VMEM((CHUNK,), jnp.int32),
            pltpu.VMEM((CHUNK, D), jnp.float32),
            pltpu.VMEM((CHUNK, D), jnp.float32),
            pltpu.SemaphoreType.DMA,
            pltpu.SemaphoreType.DMA,
        ],
    )
    def sc_kernel(x_hbm, src_hbm, dst_hbm, out_hbm,
                  idx_s, idx_d, rows_s, rows_d, sem_s, sem_d):
        wid = lax.axis_index("s") * 2 + lax.axis_index("c")

        def chunk_body(t, carry):
            base = pl.multiple_of((wid * chunks_per_worker + t) * CHUNK, CHUNK)
            pltpu.sync_copy(src_hbm.at[pl.ds(base, CHUNK)], idx_s)
            pltpu.sync_copy(dst_hbm.at[pl.ds(base, CHUNK)], idx_d)
            cp_s = pltpu.async_copy(x_hbm.at[idx_s], rows_s, sem_s)
            cp_d = pltpu.async_copy(x_hbm.at[idx_d], rows_d, sem_d)
            cp_s.wait()
            cp_d.wait()

            def row_body(r, c):
                for k in range(D // 16):
                    sl = pl.ds(k * 16, 16)
                    rows_s[r, sl] = jnp.abs(rows_s[r, sl] - rows_d[r, sl])
                return c

            lax.fori_loop(0, CHUNK, row_body, 0)
            pltpu.sync_copy(rows_s, out_hbm.at[pl.ds(base, CHUNK)])
            return carry

        lax.fori_loop(0, chunks_per_worker, chunk_body, 0)

    return sc_kernel(x, src, dst)


def _mlp_tc(dif, W1, b1, W2, b2):
    pad_e = dif.shape[0]
    be = 2048
    assert pad_e % be == 0

    def body(d_ref, w1_ref, b1_ref, w2_ref, b2_ref, o_ref):
        h = jnp.dot(d_ref[...], w1_ref[...], preferred_element_type=jnp.float32)
        h = jnp.maximum(h + b1_ref[...], 0.0)
        o_ref[...] = (
            jnp.dot(h, w2_ref[...], preferred_element_type=jnp.float32)
            + b2_ref[...]
        )

    return pl.pallas_call(
        body,
        grid=(pad_e // be,),
        in_specs=[
            pl.BlockSpec((be, D), lambda i: (i, 0)),
            pl.BlockSpec((D, H), lambda i: (0, 0)),
            pl.BlockSpec((1, H), lambda i: (0, 0)),
            pl.BlockSpec((H, 2), lambda i: (0, 0)),
            pl.BlockSpec((1, 2), lambda i: (0, 0)),
        ],
        out_specs=pl.BlockSpec((be, 2), lambda i: (i, 0)),
        out_shape=jax.ShapeDtypeStruct((pad_e, 2), jnp.float32),
    )(dif, W1, b1.reshape(1, H), W2, b2.reshape(1, 2))


def kernel(x, edge_index, W1, b1, W2, b2):
    e = edge_index.shape[1]
    pad_e = -(-e // (NW * CHUNK)) * (NW * CHUNK)
    src = jnp.pad(edge_index[0], (0, pad_e - e))
    dst = jnp.pad(edge_index[1], (0, pad_e - e))
    dif = _absdiff_sc(x, src, dst, pad_e)
    return _mlp_tc(dif, W1, b1, W2, b2)[:e]
